# baseline (speedup 1.0000x reference)
import functools
import jax
import jax.numpy as jnp
from jax import lax
from jax.experimental import pallas as pl
from jax.experimental.pallas import tpu as pltpu
from jax.experimental.pallas import tpu_sc as plsc

D = 64
L = 16
TWO_PI = 6.283185307179586


def _sin2pi(x):
    u = lax.rem(x, jnp.float32(1.0))
    u = jnp.where(u > 0.5, u - 1.0, u)
    u = jnp.where(u < -0.5, u + 1.0, u)
    u = jnp.where(u > 0.25, 0.5 - u, u)
    u = jnp.where(u < -0.25, -0.5 - u, u)
    th = jnp.float32(TWO_PI) * u
    t2 = th * th
    p = jnp.float32(2.7557319e-06)
    p = p * t2 - jnp.float32(1.9841270e-04)
    p = p * t2 + jnp.float32(8.3333333e-03)
    p = p * t2 - jnp.float32(0.16666667)
    p = p * t2 + jnp.float32(1.0)
    return th * p


def kernel(X, emb_E, emb_E_var, emb_R, emb_R_var, emb_TE, alpha_E, beta_E,
           omega_E, emb_TR, alpha_R, beta_R, omega_R):
    B = X.shape[0]
    h_i = X[:, 0]
    t_i = X[:, 1]
    r_i = X[:, 2]
    d_f = X[:, 3].astype(jnp.float32)
    NR = emb_R.shape[0]
    NE = emb_E.shape[0]
    relcat = jnp.concatenate(
        [emb_R, emb_TR, beta_R, omega_R, emb_R_var, alpha_R,
         jnp.zeros((NR, 7), jnp.float32)], axis=1)
    entcat = jnp.concatenate(
        [emb_E, emb_TE, beta_E, omega_E, emb_E_var, alpha_E,
         jnp.zeros((NE, 7), jnp.float32)], axis=1)
    RW = 328

    info = plsc.get_sparse_core_info()
    NC, NS = info.num_cores, info.num_subcores
    NW = NC * NS
    C = 64
    rows_per_w = B // NW
    n_chunks = rows_per_w // C
    JU = 4

    mesh = plsc.VectorSubcoreMesh(core_axis_name="c", subcore_axis_name="s")

    bigset = lambda: [pltpu.VMEM((C, RW), jnp.float32) for _ in range(3)]

    @functools.partial(
        pl.kernel,
        out_type=jax.ShapeDtypeStruct((B,), jnp.float32),
        mesh=mesh,
        compiler_params=pltpu.CompilerParams(
            needs_layout_passes=False, use_tc_tiling_on_sc=False),
        scratch_types=[
            pltpu.VMEM((rows_per_w,), jnp.int32),
            pltpu.VMEM((rows_per_w,), jnp.int32),
            pltpu.VMEM((rows_per_w,), jnp.int32),
            pltpu.VMEM((rows_per_w,), jnp.float32),
            pltpu.VMEM((rows_per_w,), jnp.float32),
            bigset(), bigset(),
            pltpu.SemaphoreType.DMA,
            pltpu.SemaphoreType.DMA,
            pltpu.SemaphoreType.DMA,
        ],
    )
    def score_kernel(h_hbm, t_hbm, r_hbm, d_hbm,
                     ecat, rcat,
                     out_hbm,
                     hix, tix, rix, dvb, outb,
                     set0, set1,
                     sem_i, sem0, sem1):
        wid = lax.axis_index("s") * NC + lax.axis_index("c")
        wbase = pl.multiple_of(wid * rows_per_w, rows_per_w)

        cps_i = [
            pltpu.async_copy(h_hbm.at[pl.ds(wbase, rows_per_w)], hix, sem_i),
            pltpu.async_copy(t_hbm.at[pl.ds(wbase, rows_per_w)], tix, sem_i),
            pltpu.async_copy(r_hbm.at[pl.ds(wbase, rows_per_w)], rix, sem_i),
            pltpu.async_copy(d_hbm.at[pl.ds(wbase, rows_per_w)], dvb, sem_i),
        ]
        for cp in cps_i:
            cp.wait()

        sets = (set0, set1)
        sems = (sem0, sem1)

        def fire(ci):
            s = sets[ci % 2]
            sem = sems[ci % 2]
            hslc = hix.at[pl.ds(ci * C, C)]
            tslc = tix.at[pl.ds(ci * C, C)]
            rslc = rix.at[pl.ds(ci * C, C)]
            return [
                pltpu.async_copy(ecat.at[hslc], s[0], sem),
                pltpu.async_copy(ecat.at[tslc], s[1], sem),
                pltpu.async_copy(rcat.at[rslc], s[2], sem),
            ]

        inflight = {0: fire(0)}
        for ci in range(n_chunks):
            if ci + 1 < n_chunks:
                inflight[ci + 1] = fire(ci + 1)
            for cp in inflight.pop(ci):
                cp.wait()
            s = sets[ci % 2]

            def group_body(g, _):
                off = ci * C + g * L
                rows = lax.iota(jnp.int32, L) + g * L
                d16 = dvb[pl.ds(off, L)]
                acol = jnp.full((L,), 5 * D, jnp.int32)
                dah = d16 * plsc.load_gather(s[0], [rows, acol])
                dat = d16 * plsc.load_gather(s[1], [rows, acol])
                dar = d16 * plsc.load_gather(s[2], [rows, acol])

                def jbody(j, accs):
                    new = []
                    for u in range(JU):
                        jv = jnp.full((L,), j * JU + u, jnp.int32)
                        ldh = lambda k: plsc.load_gather(s[0], [rows, jv + k * D])
                        ldt = lambda k: plsc.load_gather(s[1], [rows, jv + k * D])
                        ldr = lambda k: plsc.load_gather(s[2], [rows, jv + k * D])
                        hm = ldh(0) + dah * ldh(1) + ldh(2) * _sin2pi(ldh(3) * d16)
                        tm = ldt(0) + dat * ldt(1) + ldt(2) * _sin2pi(ldt(3) * d16)
                        rm = ldr(0) + dar * ldr(1) + ldr(2) * _sin2pi(ldr(3) * d16)
                        m = hm - tm - rm
                        m2 = m * m
                        sv = ldh(4) + ldt(4)
                        rv = ldr(4)
                        num = sv * (sv + m2) + rv * (rv + m2)
                        new.append(accs[u] + num / (rv * sv))
                    return tuple(new)

                zero = jnp.zeros((L,), jnp.float32)
                accs = lax.fori_loop(0, D // JU, jbody, (zero,) * JU)
                acc = (accs[0] + accs[1]) + (accs[2] + accs[3])
                outb[pl.ds(off, L)] = (acc - jnp.float32(2 * D)) * jnp.float32(0.25)
                return 0

            lax.fori_loop(0, C // L, group_body, 0)

        pltpu.sync_copy(outb, out_hbm.at[pl.ds(wbase, rows_per_w)])

    return score_kernel(h_i, t_i, r_i, d_f, entcat, relcat)


# speedup vs baseline: 1.1179x; 1.1179x over previous
import functools
import jax
import jax.numpy as jnp
from jax import lax
from jax.experimental import pallas as pl
from jax.experimental.pallas import tpu as pltpu
from jax.experimental.pallas import tpu_sc as plsc

D = 64
L = 16
TWO_PI = 6.283185307179586


def _sin2pi(x):
    u = lax.rem(x, jnp.float32(1.0))
    u = jnp.where(u > 0.5, u - 1.0, u)
    u = jnp.where(u < -0.5, u + 1.0, u)
    u = jnp.where(u > 0.25, 0.5 - u, u)
    u = jnp.where(u < -0.25, -0.5 - u, u)
    th = jnp.float32(TWO_PI) * u
    t2 = th * th
    p = jnp.float32(2.7557319e-06)
    p = p * t2 - jnp.float32(1.9841270e-04)
    p = p * t2 + jnp.float32(8.3333333e-03)
    p = p * t2 - jnp.float32(0.16666667)
    p = p * t2 + jnp.float32(1.0)
    return th * p


def kernel(X, emb_E, emb_E_var, emb_R, emb_R_var, emb_TE, alpha_E, beta_E,
           omega_E, emb_TR, alpha_R, beta_R, omega_R):
    B = X.shape[0]
    h_i = X[:, 0]
    t_i = X[:, 1]
    r_i = X[:, 2]
    d_f = X[:, 3].astype(jnp.float32)
    NR = emb_R.shape[0]
    emb_TEs = alpha_E * emb_TE
    relcat = jnp.concatenate(
        [emb_R, alpha_R * emb_TR, beta_R, omega_R, emb_R_var], axis=1)
    RW = 320

    info = plsc.get_sparse_core_info()
    NC, NS = info.num_cores, info.num_subcores
    NW = NC * NS
    C = 64
    rows_per_w = B // NW
    n_chunks = rows_per_w // C
    JU = 4

    mesh = plsc.VectorSubcoreMesh(core_axis_name="c", subcore_axis_name="s")

    big = lambda: pltpu.VMEM((C, D), jnp.float32)
    bigset = lambda: [big() for _ in range(10)] + [pltpu.VMEM((C, RW), jnp.float32)]

    @functools.partial(
        pl.kernel,
        out_type=jax.ShapeDtypeStruct((B,), jnp.float32),
        mesh=mesh,
        compiler_params=pltpu.CompilerParams(
            needs_layout_passes=False, use_tc_tiling_on_sc=False),
        scratch_types=[
            pltpu.VMEM((rows_per_w,), jnp.int32),
            pltpu.VMEM((rows_per_w,), jnp.int32),
            pltpu.VMEM((rows_per_w,), jnp.int32),
            pltpu.VMEM((rows_per_w,), jnp.float32),
            pltpu.VMEM((rows_per_w,), jnp.float32),
            bigset(), bigset(),
            pltpu.SemaphoreType.DMA,
            pltpu.SemaphoreType.DMA,
            pltpu.SemaphoreType.DMA,
        ],
    )
    def score_kernel(h_hbm, t_hbm, r_hbm, d_hbm,
                     eE, vE, eTE, bE, oE, rcat,
                     out_hbm,
                     hix, tix, rix, dvb, outb,
                     set0, set1,
                     sem_i, sem0, sem1):
        wid = lax.axis_index("s") * NC + lax.axis_index("c")
        wbase = pl.multiple_of(wid * rows_per_w, rows_per_w)

        cps_i = [
            pltpu.async_copy(h_hbm.at[pl.ds(wbase, rows_per_w)], hix, sem_i),
            pltpu.async_copy(t_hbm.at[pl.ds(wbase, rows_per_w)], tix, sem_i),
            pltpu.async_copy(r_hbm.at[pl.ds(wbase, rows_per_w)], rix, sem_i),
            pltpu.async_copy(d_hbm.at[pl.ds(wbase, rows_per_w)], dvb, sem_i),
        ]
        for cp in cps_i:
            cp.wait()

        sets = (set0, set1)
        sems = (sem0, sem1)

        def fire(ci):
            s = sets[ci % 2]
            sem = sems[ci % 2]
            hslc = hix.at[pl.ds(ci * C, C)]
            tslc = tix.at[pl.ds(ci * C, C)]
            rslc = rix.at[pl.ds(ci * C, C)]
            return [
                pltpu.async_copy(eE.at[hslc], s[0], sem),
                pltpu.async_copy(eTE.at[hslc], s[1], sem),
                pltpu.async_copy(bE.at[hslc], s[2], sem),
                pltpu.async_copy(oE.at[hslc], s[3], sem),
                pltpu.async_copy(vE.at[hslc], s[4], sem),
                pltpu.async_copy(eE.at[tslc], s[5], sem),
                pltpu.async_copy(eTE.at[tslc], s[6], sem),
                pltpu.async_copy(bE.at[tslc], s[7], sem),
                pltpu.async_copy(oE.at[tslc], s[8], sem),
                pltpu.async_copy(vE.at[tslc], s[9], sem),
                pltpu.async_copy(rcat.at[rslc], s[10], sem),
            ]

        inflight = {0: fire(0)}
        for ci in range(n_chunks):
            if ci + 1 < n_chunks:
                inflight[ci + 1] = fire(ci + 1)
            for cp in inflight.pop(ci):
                cp.wait()
            s = sets[ci % 2]

            def group_body(g, _):
                off = ci * C + g * L
                rows = lax.iota(jnp.int32, L) + g * L
                d16 = dvb[pl.ds(off, L)]

                def jbody(j, accs):
                    new = []
                    for u in range(JU):
                        jv = jnp.full((L,), j * JU + u, jnp.int32)
                        ld = lambda k: plsc.load_gather(s[k], [rows, jv])
                        ldr = lambda k: plsc.load_gather(s[10], [rows, jv + k * D])
                        hm = ld(0) + d16 * ld(1) + ld(2) * _sin2pi(ld(3) * d16)
                        tm = ld(5) + d16 * ld(6) + ld(7) * _sin2pi(ld(8) * d16)
                        rm = ldr(0) + d16 * ldr(1) + ldr(2) * _sin2pi(ldr(3) * d16)
                        m = hm - tm - rm
                        m2 = m * m
                        sv = ld(4) + ld(9)
                        rv = ldr(4)
                        num = sv * (sv + m2) + rv * (rv + m2)
                        new.append(accs[u] + num / (rv * sv))
                    return tuple(new)

                zero = jnp.zeros((L,), jnp.float32)
                accs = lax.fori_loop(0, D // JU, jbody, (zero,) * JU)
                acc = (accs[0] + accs[1]) + (accs[2] + accs[3])
                outb[pl.ds(off, L)] = (acc - jnp.float32(2 * D)) * jnp.float32(0.25)
                return 0

            lax.fori_loop(0, C // L, group_body, 0)

        pltpu.sync_copy(outb, out_hbm.at[pl.ds(wbase, rows_per_w)])

    return score_kernel(h_i, t_i, r_i, d_f,
                        emb_E, emb_E_var, emb_TEs, beta_E, omega_E, relcat)


# speedup vs baseline: 1.3371x; 1.1962x over previous
import functools
import jax
import jax.numpy as jnp
from jax import lax
from jax.experimental import pallas as pl
from jax.experimental.pallas import tpu as pltpu
from jax.experimental.pallas import tpu_sc as plsc

D = 64
L = 16
TWO_PI = 6.283185307179586


def _sin2pi(x):
    u = lax.rem(x, jnp.float32(1.0))
    u = jnp.where(u > 0.5, u - 1.0, u)
    u = jnp.where(u < -0.5, u + 1.0, u)
    u = jnp.where(u > 0.25, 0.5 - u, u)
    u = jnp.where(u < -0.25, -0.5 - u, u)
    th = jnp.float32(TWO_PI) * u
    t2 = th * th
    p = jnp.float32(2.7557319e-06)
    p = p * t2 - jnp.float32(1.9841270e-04)
    p = p * t2 + jnp.float32(8.3333333e-03)
    p = p * t2 - jnp.float32(0.16666667)
    p = p * t2 + jnp.float32(1.0)
    return th * p


def kernel(X, emb_E, emb_E_var, emb_R, emb_R_var, emb_TE, alpha_E, beta_E,
           omega_E, emb_TR, alpha_R, beta_R, omega_R):
    B = X.shape[0]
    h_i = X[:, 0]
    t_i = X[:, 1]
    r_i = X[:, 2]
    d_f = X[:, 3].astype(jnp.float32)
    alpha_E1 = alpha_E.reshape(-1)
    NR = emb_R.shape[0]
    relcat = jnp.concatenate(
        [emb_R, emb_TR, beta_R, omega_R, emb_R_var, alpha_R,
         jnp.zeros((NR, 7), jnp.float32)], axis=1)
    RW = 328

    info = plsc.get_sparse_core_info()
    NC, NS = info.num_cores, info.num_subcores
    NW = NC * NS
    C = 64
    rows_per_w = B // NW
    n_chunks = rows_per_w // C
    JU = 4

    mesh = plsc.VectorSubcoreMesh(core_axis_name="c", subcore_axis_name="s")

    big = lambda: pltpu.VMEM((C, D), jnp.float32)
    bigset = lambda: [big() for _ in range(10)] + [pltpu.VMEM((C, RW), jnp.float32)]

    @functools.partial(
        pl.kernel,
        out_type=jax.ShapeDtypeStruct((B,), jnp.float32),
        mesh=mesh,
        compiler_params=pltpu.CompilerParams(
            needs_layout_passes=False, use_tc_tiling_on_sc=False),
        scratch_types=[
            pltpu.VMEM((rows_per_w,), jnp.int32),
            pltpu.VMEM((rows_per_w,), jnp.int32),
            pltpu.VMEM((rows_per_w,), jnp.int32),
            pltpu.VMEM((rows_per_w,), jnp.float32),
            pltpu.VMEM((rows_per_w,), jnp.float32),
            bigset(), bigset(),
            [pltpu.VMEM((C,), jnp.float32) for _ in range(2)],
            [pltpu.VMEM((C,), jnp.float32) for _ in range(2)],
            pltpu.SemaphoreType.DMA,
            pltpu.SemaphoreType.DMA,
            pltpu.SemaphoreType.DMA,
        ],
    )
    def score_kernel(h_hbm, t_hbm, r_hbm, d_hbm,
                     eE, vE, eTE, aE, bE, oE, rcat,
                     out_hbm,
                     hix, tix, rix, dvb, outb,
                     set0, set1, al0, al1,
                     sem_i, sem0, sem1):
        wid = lax.axis_index("s") * NC + lax.axis_index("c")
        wbase = pl.multiple_of(wid * rows_per_w, rows_per_w)

        cps_i = [
            pltpu.async_copy(h_hbm.at[pl.ds(wbase, rows_per_w)], hix, sem_i),
            pltpu.async_copy(t_hbm.at[pl.ds(wbase, rows_per_w)], tix, sem_i),
            pltpu.async_copy(r_hbm.at[pl.ds(wbase, rows_per_w)], rix, sem_i),
            pltpu.async_copy(d_hbm.at[pl.ds(wbase, rows_per_w)], dvb, sem_i),
        ]
        for cp in cps_i:
            cp.wait()

        sets = (set0, set1)
        als = (al0, al1)
        sems = (sem0, sem1)

        def fire(ci):
            s = sets[ci % 2]
            a = als[ci % 2]
            sem = sems[ci % 2]
            hslc = hix.at[pl.ds(ci * C, C)]
            tslc = tix.at[pl.ds(ci * C, C)]
            rslc = rix.at[pl.ds(ci * C, C)]
            return [
                pltpu.async_copy(eE.at[hslc], s[0], sem),
                pltpu.async_copy(eTE.at[hslc], s[1], sem),
                pltpu.async_copy(bE.at[hslc], s[2], sem),
                pltpu.async_copy(oE.at[hslc], s[3], sem),
                pltpu.async_copy(vE.at[hslc], s[4], sem),
                pltpu.async_copy(eE.at[tslc], s[5], sem),
                pltpu.async_copy(eTE.at[tslc], s[6], sem),
                pltpu.async_copy(bE.at[tslc], s[7], sem),
                pltpu.async_copy(oE.at[tslc], s[8], sem),
                pltpu.async_copy(vE.at[tslc], s[9], sem),
                pltpu.async_copy(rcat.at[rslc], s[10], sem),
                pltpu.async_copy(aE.at[hslc], a[0], sem),
                pltpu.async_copy(aE.at[tslc], a[1], sem),
            ]

        inflight = {0: fire(0)}
        for ci in range(n_chunks):
            if ci + 1 < n_chunks:
                inflight[ci + 1] = fire(ci + 1)
            for cp in inflight.pop(ci):
                cp.wait()
            s = sets[ci % 2]
            a = als[ci % 2]

            def group_body(g, _):
                off = ci * C + g * L
                rows = lax.iota(jnp.int32, L) + g * L
                d16 = dvb[pl.ds(off, L)]
                dah = d16 * a[0][pl.ds(g * L, L)]
                dat = d16 * a[1][pl.ds(g * L, L)]
                dar = d16 * plsc.load_gather(
                    s[10], [rows, jnp.full((L,), 5 * D, jnp.int32)])

                def jbody(j, accs):
                    new = []
                    for u in range(JU):
                        jv = jnp.full((L,), j * JU + u, jnp.int32)
                        ld = lambda k: plsc.load_gather(s[k], [rows, jv])
                        ldr = lambda k: plsc.load_gather(s[10], [rows, jv + k * D])
                        hm = ld(0) + dah * ld(1) + ld(2) * _sin2pi(ld(3) * d16)
                        tm = ld(5) + dat * ld(6) + ld(7) * _sin2pi(ld(8) * d16)
                        rm = ldr(0) + dar * ldr(1) + ldr(2) * _sin2pi(ldr(3) * d16)
                        m = hm - tm - rm
                        m2 = m * m
                        sv = ld(4) + ld(9)
                        rv = ldr(4)
                        num = sv * (sv + m2) + rv * (rv + m2)
                        new.append(accs[u] + num / (rv * sv))
                    return tuple(new)

                zero = jnp.zeros((L,), jnp.float32)
                accs = lax.fori_loop(0, D // JU, jbody, (zero,) * JU)
                acc = (accs[0] + accs[1]) + (accs[2] + accs[3])
                outb[pl.ds(off, L)] = (acc - jnp.float32(2 * D)) * jnp.float32(0.25)
                return 0

            lax.fori_loop(0, C // L, group_body, 0)

        pltpu.sync_copy(outb, out_hbm.at[pl.ds(wbase, rows_per_w)])

    return score_kernel(h_i, t_i, r_i, d_f,
                        emb_E, emb_E_var, emb_TE, alpha_E1, beta_E, omega_E,
                        relcat)
